# baseline (speedup 1.0000x reference)
import functools

import jax
import jax.numpy as jnp
from jax import lax
from jax.experimental import pallas as pl
from jax.experimental.pallas import tpu as pltpu
from jax.experimental.pallas import tpu_sc as plsc

N = 10000
K = 32
E = N * K
H = 64
NB_E = 200
GRID_E = N // NB_E
NB_N = 2000
GRID_N = N // NB_N
CH = 512
NCHUNK = E // CH
SUB = CH // 128
NW = 32
PER_W = -(-NCHUNK // NW)
DEC_PAD = 512


def _sp(x):
    return jnp.maximum(x, 0.0) + jnp.log1p(jnp.exp(-jnp.abs(x)))


def _sc_gather(table, idx3d):
    mesh = plsc.VectorSubcoreMesh(core_axis_name="c", subcore_axis_name="s")
    nch = idx3d.shape[0]
    m = nch * CH
    trips = -(-nch // NW)

    @functools.partial(
        pl.kernel, mesh=mesh,
        out_type=jax.ShapeDtypeStruct((m, 2 * H), jnp.float32),
        scratch_types=[
            pltpu.VMEM((SUB, 128), jnp.int32),
            pltpu.VMEM((CH, 2 * H), jnp.float32),
            pltpu.SemaphoreType.DMA,
        ],
    )
    def k(table_hbm, idx_hbm, out_hbm, idx_v, rows_v, sem):
        wid = lax.axis_index("s") * 2 + lax.axis_index("c")

        def body(t, carry):
            cid = wid + NW * t

            @pl.when(cid < nch)
            def _():
                pltpu.sync_copy(idx_hbm.at[cid], idx_v)
                cps = [
                    pltpu.async_copy(table_hbm.at[idx_v.at[j]],
                                     rows_v.at[pl.ds(j * 128, 128)], sem)
                    for j in range(SUB)
                ]
                for cp in cps:
                    cp.wait()
                pltpu.sync_copy(rows_v, out_hbm.at[pl.ds(cid * CH, CH)])

            return carry

        lax.fori_loop(0, trips, body, 0)

    return k(table, idx3d)


def _cumsum_kernel(srt, par, sume):
    blk = NB_E * K

    def body(srt_r, par_r, sume_r, s_o, carry):
        i = pl.program_id(0)

        @pl.when(i == 0)
        def _():
            carry[...] = jnp.zeros((1, H), jnp.float32)

        sel = (jnp.where(par_r[...] > 0.5, srt_r[:, H:], srt_r[:, :H])
               - sume_r[...] * (1.0 / E))
        s = sel
        sh = 1
        while sh < blk:
            s = s + jnp.concatenate(
                [jnp.zeros((sh, H), jnp.float32), s[:-sh]], axis=0)
            sh *= 2
        s = s + carry[...]
        s_o[...] = s
        carry[...] = s[blk - 1:blk, :]

    return pl.pallas_call(
        body, grid=(GRID_E,),
        in_specs=[pl.BlockSpec((blk, 2 * H), lambda i: (i, 0)),
                  pl.BlockSpec((blk, 1), lambda i: (i, 0)),
                  _full((1, H))],
        out_specs=pl.BlockSpec((blk, H), lambda i: (i, 0)),
        out_shape=jax.ShapeDtypeStruct((E, H), jnp.float32),
        scratch_shapes=[pltpu.VMEM((1, H), jnp.float32)],
    )(srt, par, sume)


def _full(shape):
    return pl.BlockSpec(shape, lambda i: tuple(0 for _ in shape))


def _prep_kernel(nodes, g2d, wn1, bn1, wn2, bn2, wg1, bg1, wg2, bg2,
                 wp1, wq1, wcg1, b1e1, wgnb1, b1n1, wggb1, b1g1):
    def body(nodes_r, g_r, wn1_r, bn1_r, wn2_r, bn2_r, wg1_r, bg1_r, wg2_r,
             bg2_r, wp1_r, wq1_r, wcg1_r, b1e1_r, wgnb1_r, b1n1_r, wggb1_r,
             b1g1_r, enc_n_o, p1_o, q1_o, enc_g_o, cg1_o, gnb1_o, ggb1_o):
        enc_n = _sp(_sp(nodes_r[...] @ wn1_r[...] + bn1_r[...])
                    @ wn2_r[...] + bn2_r[...])
        enc_g = _sp(_sp(g_r[...] @ wg1_r[...] + bg1_r[...])
                    @ wg2_r[...] + bg2_r[...])
        enc_n_o[...] = enc_n
        p1_o[:, :H] = enc_n @ wp1_r[...]
        p1_o[:, H:] = jnp.zeros((N, H), jnp.float32)
        q1_o[...] = enc_n @ wq1_r[...]
        enc_g_o[...] = enc_g
        cg1_o[...] = enc_g @ wcg1_r[...] + b1e1_r[...]
        gnb1_o[...] = enc_g @ wgnb1_r[...] + b1n1_r[...]
        ggb1_o[...] = enc_g @ wggb1_r[...] + b1g1_r[...]

    out_shape = [
        jax.ShapeDtypeStruct((N, H), jnp.float32),
        jax.ShapeDtypeStruct((N, 2 * H), jnp.float32),
        jax.ShapeDtypeStruct((N, H), jnp.float32),
        jax.ShapeDtypeStruct((1, H), jnp.float32),
        jax.ShapeDtypeStruct((1, H), jnp.float32),
        jax.ShapeDtypeStruct((1, H), jnp.float32),
        jax.ShapeDtypeStruct((1, H), jnp.float32),
    ]
    return pl.pallas_call(body, out_shape=out_shape)(
        nodes, g2d, wn1, bn1, wn2, bn2, wg1, bg1, wg2, bg2,
        wp1, wq1, wcg1, b1e1, wgnb1, b1n1, wggb1, b1g1)


def _edge_kernel(edges_flat, g_rows, q, cg, we1, be1, we2, be2,
                 wab, w2e, b2e, out_e=None, wa=None):
    two_e = out_e is not None

    def body(*refs):
        if two_e:
            (edges_r, g_r, q_r, cg_r, we1_r, be1_r, we2_r, be2_r, wab_r,
             w2e_r, b2e_r, oute_r, wa_r, enew_o, inc_o, sume_o) = refs
        else:
            (edges_r, g_r, q_r, cg_r, we1_r, be1_r, we2_r, be2_r, wab_r,
             w2e_r, b2e_r, enew_o, inc_o, sume_o) = refs
        i = pl.program_id(0)
        enc_e = _sp(_sp(edges_r[...] @ we1_r[...] + be1_r[...])
                    @ we2_r[...] + be2_r[...])
        z = enc_e @ wab_r[...] + g_r[:, :H] + cg_r[...]
        if two_e:
            z = z + oute_r[...] @ wa_r[...]
        qb = jnp.broadcast_to(q_r[...][:, None, :], (NB_E, K, H))
        z = z + qb.reshape(NB_E * K, H)
        en = _sp(_sp(z) @ w2e_r[...] + b2e_r[...])
        enew_o[...] = en.reshape(NB_E * K // 2, 2, H)
        inc_o[...] = en.reshape(NB_E, K, H).sum(axis=1)
        s = en.sum(axis=0, keepdims=True)

        @pl.when(i == 0)
        def _():
            sume_o[...] = s

        @pl.when(i > 0)
        def _():
            sume_o[...] += s

    blk = NB_E * K
    in_specs = [
        pl.BlockSpec((blk, 4), lambda i: (i, 0)),
        pl.BlockSpec((blk, 2 * H), lambda i: (i, 0)),
        pl.BlockSpec((NB_E, H), lambda i: (i, 0)),
        _full((1, H)), _full((4, H)), _full((1, H)), _full((H, H)),
        _full((1, H)), _full((H, H)), _full((H, H)), _full((1, H)),
    ]
    args = [edges_flat, g_rows, q, cg, we1, be1, we2, be2, wab, w2e, b2e]
    if two_e:
        in_specs += [pl.BlockSpec((blk, H), lambda i: (i, 0)), _full((H, H))]
        args += [out_e, wa]
    out_shape = [
        jax.ShapeDtypeStruct((E // 2, 2, H), jnp.float32),
        jax.ShapeDtypeStruct((N, H), jnp.float32),
        jax.ShapeDtypeStruct((1, H), jnp.float32),
    ]
    out_specs = [
        pl.BlockSpec((blk // 2, 2, H), lambda i: (i, 0, 0)),
        pl.BlockSpec((NB_E, H), lambda i: (i, 0)),
        _full((1, H)),
    ]
    return pl.pallas_call(
        body, grid=(GRID_E,), in_specs=in_specs, out_specs=out_specs,
        out_shape=out_shape)(*args)


def _node_kernel(out_n, enc_n, inc, bnd, gnb, wna, wnb, wnc, wnd, w2n, b2n,
                 sum_e, ggb, wgn, wge, w2g, b2g, enc_g, nxt):
    final = len(nxt) == 6

    def body(*refs):
        (outn_r, encn_r, inc_r, ghi_r, glo_r, phi_r, plo_r, mhi_r, mlo_r,
         deg_r, gnb_r, wna_r, wnb_r, wnc_r,
         wnd_r, w2n_r, b2n_r, sume_r, ggb_r, wgn_r, wge_r, w2g_r, b2g_r,
         encg_r) = refs[:24]
        nxt_r = refs[24:24 + len(nxt)]
        outs = refs[24 + len(nxt):]
        i = pl.program_id(0)

        outg = (jnp.where(phi_r[...] > 0.5, ghi_r[:, H:], ghi_r[:, :H])
                * mhi_r[...]
                - jnp.where(plo_r[...] > 0.5, glo_r[:, H:], glo_r[:, :H])
                * mlo_r[...]
                + deg_r[...] * (sume_r[...] * (1.0 / E)))
        zn = (outn_r[...] @ wna_r[...] + encn_r[...] @ wnb_r[...]
              + inc_r[...] @ wnc_r[...]
              + outg @ wnd_r[...] + gnb_r[...])
        nn = _sp(_sp(zn) @ w2n_r[...] + b2n_r[...])
        if final:
            sumn_o, dec_o = outs
        else:
            (nnew_o, p2_o, q2_o, sumn_o, gnew_o, cg2_o, gnb2_o, ggb2_o) = outs
            nnew_o[...] = nn
            p2_o[:, :H] = nn @ nxt_r[0][...] + encn_r[...] @ nxt_r[1][...]
            p2_o[:, H:] = jnp.zeros((NB_N, H), jnp.float32)
            q2_o[...] = nn @ nxt_r[2][...] + encn_r[...] @ nxt_r[3][...]
        s = nn.sum(axis=0, keepdims=True)

        @pl.when(i == 0)
        def _():
            sumn_o[...] = s

        @pl.when(i > 0)
        def _():
            sumn_o[...] += s

        @pl.when(i == GRID_N - 1)
        def _():
            zg = (ggb_r[...] + sumn_o[...] @ wgn_r[...]
                  + sume_r[...] @ wge_r[...])
            gn = _sp(_sp(zg) @ w2g_r[...] + b2g_r[...])
            if final:
                d1, db1, d2, db2, d3, db3 = nxt_r
                d = _sp(gn @ d1[...] + db1[...])
                d = _sp(d @ d2[...] + db2[...])
                dec_o[...] = d @ d3[...] + db3[...]
            else:
                gnew_o[...] = gn
                cg2_o[...] = gn @ nxt_r[4][...] + encg_r[...] @ nxt_r[5][...] + nxt_r[6][...]
                gnb2_o[...] = gn @ nxt_r[7][...] + encg_r[...] @ nxt_r[8][...] + nxt_r[9][...]
                ggb2_o[...] = gn @ nxt_r[10][...] + encg_r[...] @ nxt_r[11][...] + nxt_r[12][...]

    nblk = pl.BlockSpec((NB_N, H), lambda i: (i, 0))
    gblk = pl.BlockSpec((NB_N, 2 * H), lambda i: (i, 0))
    pblk = pl.BlockSpec((NB_N, 1), lambda i: (i, 0))
    in_specs = [
        nblk, nblk, nblk,
        gblk, gblk, pblk, pblk, pblk, pblk, pblk,
        _full((1, H)), _full((H, H)), _full((H, H)), _full((H, H)),
        _full((H, H)), _full((H, H)), _full((1, H)),
        _full((1, H)), _full((1, H)), _full((H, H)), _full((H, H)),
        _full((H, H)), _full((1, H)), _full((1, H)),
    ]
    for w in nxt:
        in_specs.append(_full(tuple(w.shape)))
    if final:
        out_shape = [
            jax.ShapeDtypeStruct((1, H), jnp.float32),
            jax.ShapeDtypeStruct((1, DEC_PAD), jnp.float32),
        ]
        out_specs = [_full((1, H)), _full((1, DEC_PAD))]
    else:
        out_shape = [
            jax.ShapeDtypeStruct((N, H), jnp.float32),
            jax.ShapeDtypeStruct((N, 2 * H), jnp.float32),
            jax.ShapeDtypeStruct((N, H), jnp.float32),
            jax.ShapeDtypeStruct((1, H), jnp.float32),
            jax.ShapeDtypeStruct((1, H), jnp.float32),
            jax.ShapeDtypeStruct((1, H), jnp.float32),
            jax.ShapeDtypeStruct((1, H), jnp.float32),
            jax.ShapeDtypeStruct((1, H), jnp.float32),
        ]
        out_specs = ([nblk, pl.BlockSpec((NB_N, 2 * H), lambda i: (i, 0)),
                      nblk] + [_full((1, H))] * 5)
    ghi, glo, phi, plo, mhi, mlo, deg = bnd
    return pl.pallas_call(
        body, grid=(GRID_N,), in_specs=in_specs, out_specs=out_specs,
        out_shape=out_shape,
    )(out_n, enc_n, inc, ghi, glo, phi, plo, mhi, mlo, deg, gnb, wna, wnb,
      wnc, wnd, w2n, b2n, sum_e, ggb, wgn, wge, w2g, b2g, enc_g, *nxt)


def kernel(nodes, edges, globals_, edge_idx, params):
    f32 = jnp.float32
    i32 = jnp.int32
    idx3d = edge_idx.reshape(NCHUNK, SUB, 128)
    idx_flat = edge_idx.reshape(E)
    edges_flat = edges.reshape(E, 4)
    g2d = globals_.reshape(1, -1)

    perm = jnp.argsort(idx_flat)
    sorted_idx = idx_flat[perm]
    b = jnp.searchsorted(sorted_idx, jnp.arange(N + 1, dtype=i32)).astype(i32)
    gperm3d = (perm // 2).astype(i32).reshape(NCHUNK, SUB, 128)
    spar = (perm % 2).astype(f32).reshape(E, 1)
    lo = jnp.maximum(b[:-1] - 1, 0)
    hi = jnp.maximum(b[1:] - 1, 0)
    PADN = 16384
    lo3d = jnp.zeros((PADN,), i32).at[:N].set(lo // 2).reshape(
        PADN // CH, SUB, 128)
    hi3d = jnp.zeros((PADN,), i32).at[:N].set(hi // 2).reshape(
        PADN // CH, SUB, 128)
    plo = (lo % 2).astype(f32).reshape(N, 1)
    phi = (hi % 2).astype(f32).reshape(N, 1)
    mlo = (b[:-1] > 0).astype(f32).reshape(N, 1)
    mhi = (b[1:] > 0).astype(f32).reshape(N, 1)
    deg = (b[1:] - b[:-1]).astype(f32).reshape(N, 1)

    def outgoing_bnd(e3, sume):
        srt = _sc_gather(e3.reshape(E // 2, 2 * H), gperm3d)
        s = _cumsum_kernel(srt, spar, sume)
        stab = s.reshape(E // 2, 2 * H)
        return (_sc_gather(stab, hi3d), _sc_gather(stab, lo3d),
                phi, plo, mhi, mlo, deg)

    def row2d(b):
        return b.reshape(1, -1).astype(f32)

    (wn1, bn1), (wn2, bn2) = params['node_enc']
    (we1, be1), (we2, be2) = params['edge_enc']
    (wg1, bg1), (wg2, bg2) = params['glob_enc']
    bn1, bn2, bg1, bg2, be1, be2 = map(row2d, (bn1, bn2, bg1, bg2, be1, be2))

    rblk = []
    for rp in params['rec']:
        (w1e, b1e), (w2e, b2e) = rp['edge_fn']
        (w1n, b1n), (w2n, b2n) = rp['node_fn']
        (w1g, b1g), (w2g, b2g) = rp['glob_fn']
        rblk.append(dict(
            wa=w1e[0:64], wb=w1e[64:128],
            wp_a=w1e[128:192], wp_b=w1e[192:256],
            wq_a=w1e[256:320], wq_b=w1e[320:384],
            wcg_a=w1e[384:448], wcg_b=w1e[448:512],
            b1e=row2d(b1e), w2e=w2e, b2e=row2d(b2e),
            wna=w1n[0:64], wnb=w1n[64:128], wnc=w1n[128:192],
            wnd=w1n[192:256], wgnb_a=w1n[256:320], wgnb_b=w1n[320:384],
            b1n=row2d(b1n), w2n=w2n, b2n=row2d(b2n),
            wgg_a=w1g[0:64], wgg_b=w1g[64:128], wgn=w1g[128:192],
            wge=w1g[192:256], b1g=row2d(b1g), w2g=w2g, b2g=row2d(b2g)))
    r1, r2 = rblk

    (d1, db1), (d2, db2), (d3, db3) = params['decoder']
    d3p = jnp.zeros((H, DEC_PAD), f32).at[:, :d3.shape[1]].set(d3)
    db3p = jnp.zeros((1, DEC_PAD), f32).at[0, :d3.shape[1]].set(db3)

    enc_n, p1, q1, enc_g, cg1, gnb1, ggb1 = _prep_kernel(
        nodes, g2d, wn1, bn1, wn2, bn2, wg1, bg1, wg2, bg2,
        r1['wp_a'] + r1['wp_b'], r1['wq_a'] + r1['wq_b'],
        r1['wcg_a'] + r1['wcg_b'], r1['b1e'],
        r1['wgnb_a'] + r1['wgnb_b'], r1['b1n'],
        r1['wgg_a'] + r1['wgg_b'], r1['b1g'])

    g1_rows = _sc_gather(p1, idx3d)
    e_new1, inc1, sum_e1 = _edge_kernel(
        edges_flat, g1_rows, q1, cg1, we1, be1, we2, be2,
        r1['wa'] + r1['wb'], r1['w2e'], r1['b2e'])
    bnd1 = outgoing_bnd(e_new1, sum_e1)
    nxt = (r2['wp_a'], r2['wp_b'], r2['wq_a'], r2['wq_b'],
           r2['wcg_a'], r2['wcg_b'], r2['b1e'],
           r2['wgnb_a'], r2['wgnb_b'], r2['b1n'],
           r2['wgg_a'], r2['wgg_b'], r2['b1g'])
    (n_new1, p2, q2, _sum_n1, g_new1, cg2, gnb2, ggb2) = _node_kernel(
        enc_n, enc_n, inc1, bnd1, gnb1,
        r1['wna'] + r1['wnb'], jnp.zeros((H, H), f32), r1['wnc'], r1['wnd'],
        r1['w2n'], r1['b2n'], sum_e1, ggb1, r1['wgn'], r1['wge'],
        r1['w2g'], r1['b2g'], enc_g, nxt)

    g2_rows = _sc_gather(p2, idx3d)
    e_new2, inc2, sum_e2 = _edge_kernel(
        edges_flat, g2_rows, q2, cg2, we1, be1, we2, be2,
        r2['wb'], r2['w2e'], r2['b2e'],
        out_e=e_new1.reshape(E, H), wa=r2['wa'])
    bnd2 = outgoing_bnd(e_new2, sum_e2)
    dec = (d1, row2d(db1), d2, row2d(db2), d3p, db3p)
    _sum_n2, out = _node_kernel(
        n_new1, enc_n, inc2, bnd2, gnb2,
        r2['wna'], r2['wnb'], r2['wnc'], r2['wnd'],
        r2['w2n'], r2['b2n'], sum_e2, ggb2, r2['wgn'], r2['wge'],
        r2['w2g'], r2['b2g'], enc_g, dec)

    return out[0, :d3.shape[1]]


# speedup vs baseline: 1.1187x; 1.1187x over previous
import functools

import jax
import jax.numpy as jnp
from jax import lax
from jax.experimental import pallas as pl
from jax.experimental.pallas import tpu as pltpu
from jax.experimental.pallas import tpu_sc as plsc

N = 10000
K = 32
E = N * K
H = 64
NB_E = 200
GRID_E = N // NB_E
NB_N = 2000
GRID_N = N // NB_N
CH = 512
NCHUNK = E // CH
SUB = CH // 128
NW = 32
PER_W = -(-NCHUNK // NW)
DEC_PAD = 512


def _sp(x):
    return jnp.maximum(x, 0.0) + jnp.log1p(jnp.exp(-jnp.abs(x)))


def _sc_gather(table, idx3d):
    mesh = plsc.VectorSubcoreMesh(core_axis_name="c", subcore_axis_name="s")
    nch = idx3d.shape[0]
    m = nch * CH
    trips = -(-nch // NW)

    @functools.partial(
        pl.kernel, mesh=mesh,
        out_type=jax.ShapeDtypeStruct((m, 2 * H), jnp.float32),
        scratch_types=[
            pltpu.VMEM((SUB, 128), jnp.int32),
            pltpu.VMEM((CH, 2 * H), jnp.float32),
            pltpu.SemaphoreType.DMA,
        ],
    )
    def k(table_hbm, idx_hbm, out_hbm, idx_v, rows_v, sem):
        wid = lax.axis_index("s") * 2 + lax.axis_index("c")

        def body(t, carry):
            cid = wid + NW * t

            @pl.when(cid < nch)
            def _():
                pltpu.sync_copy(idx_hbm.at[cid], idx_v)
                cps = [
                    pltpu.async_copy(table_hbm.at[idx_v.at[j]],
                                     rows_v.at[pl.ds(j * 128, 128)], sem)
                    for j in range(SUB)
                ]
                for cp in cps:
                    cp.wait()
                pltpu.sync_copy(rows_v, out_hbm.at[pl.ds(cid * CH, CH)])

            return carry

        lax.fori_loop(0, trips, body, 0)

    return k(table, idx3d)


def _cumsum_kernel(srt, par, sume, tri):
    blk = NB_E * K

    def body(srt_r, par_r, sume_r, tri_r, s_o, carry):
        i = pl.program_id(0)

        @pl.when(i == 0)
        def _():
            carry[...] = jnp.zeros((1, H), jnp.float32)

        sel = (jnp.where(par_r[...] > 0.5, srt_r[:, H:], srt_r[:, :H])
               - sume_r[...] * (1.0 / E))
        x3 = sel.reshape(blk // 32, 32, H)
        f = x3
        sh = 1
        while sh < 32:
            f = f + jnp.concatenate(
                [jnp.zeros((blk // 32, sh, H), jnp.float32), f[:, :-sh]],
                axis=1)
            sh *= 2
        coarse = tri_r[...] @ x3.sum(axis=1)
        s = (f + coarse[:, None, :]).reshape(blk, H) + carry[...]
        s_o[...] = s
        carry[...] = s[blk - 1:blk, :]

    return pl.pallas_call(
        body, grid=(GRID_E,),
        in_specs=[pl.BlockSpec((blk, 2 * H), lambda i: (i, 0)),
                  pl.BlockSpec((blk, 1), lambda i: (i, 0)),
                  _full((1, H)), _full((NB_E, NB_E))],
        out_specs=pl.BlockSpec((blk, H), lambda i: (i, 0)),
        out_shape=jax.ShapeDtypeStruct((E, H), jnp.float32),
        scratch_shapes=[pltpu.VMEM((1, H), jnp.float32)],
    )(srt, par, sume, tri)


def _full(shape):
    return pl.BlockSpec(shape, lambda i: tuple(0 for _ in shape))


def _prep_kernel(nodes, g2d, wn1, bn1, wn2, bn2, wg1, bg1, wg2, bg2,
                 wp1, wq1, wcg1, b1e1, wgnb1, b1n1, wggb1, b1g1):
    def body(nodes_r, g_r, wn1_r, bn1_r, wn2_r, bn2_r, wg1_r, bg1_r, wg2_r,
             bg2_r, wp1_r, wq1_r, wcg1_r, b1e1_r, wgnb1_r, b1n1_r, wggb1_r,
             b1g1_r, enc_n_o, p1_o, q1_o, enc_g_o, cg1_o, gnb1_o, ggb1_o):
        enc_n = _sp(_sp(nodes_r[...] @ wn1_r[...] + bn1_r[...])
                    @ wn2_r[...] + bn2_r[...])
        enc_g = _sp(_sp(g_r[...] @ wg1_r[...] + bg1_r[...])
                    @ wg2_r[...] + bg2_r[...])
        enc_n_o[...] = enc_n
        p1_o[:, :H] = enc_n @ wp1_r[...]
        p1_o[:, H:] = jnp.zeros((N, H), jnp.float32)
        q1_o[...] = enc_n @ wq1_r[...]
        enc_g_o[...] = enc_g
        cg1_o[...] = enc_g @ wcg1_r[...] + b1e1_r[...]
        gnb1_o[...] = enc_g @ wgnb1_r[...] + b1n1_r[...]
        ggb1_o[...] = enc_g @ wggb1_r[...] + b1g1_r[...]

    out_shape = [
        jax.ShapeDtypeStruct((N, H), jnp.float32),
        jax.ShapeDtypeStruct((N, 2 * H), jnp.float32),
        jax.ShapeDtypeStruct((N, H), jnp.float32),
        jax.ShapeDtypeStruct((1, H), jnp.float32),
        jax.ShapeDtypeStruct((1, H), jnp.float32),
        jax.ShapeDtypeStruct((1, H), jnp.float32),
        jax.ShapeDtypeStruct((1, H), jnp.float32),
    ]
    return pl.pallas_call(body, out_shape=out_shape)(
        nodes, g2d, wn1, bn1, wn2, bn2, wg1, bg1, wg2, bg2,
        wp1, wq1, wcg1, b1e1, wgnb1, b1n1, wggb1, b1g1)


def _edge_kernel(edges_flat, g_rows, q, cg, we1, be1, we2, be2,
                 wab, w2e, b2e, out_e=None, wa=None):
    two_e = out_e is not None

    def body(*refs):
        if two_e:
            (edges_r, g_r, q_r, cg_r, we1_r, be1_r, we2_r, be2_r, wab_r,
             w2e_r, b2e_r, oute_r, wa_r, enew_o, inc_o, sume_o) = refs
        else:
            (edges_r, g_r, q_r, cg_r, we1_r, be1_r, we2_r, be2_r, wab_r,
             w2e_r, b2e_r, enew_o, inc_o, sume_o) = refs
        i = pl.program_id(0)
        enc_e = _sp(_sp(edges_r[...] @ we1_r[...] + be1_r[...])
                    @ we2_r[...] + be2_r[...])
        z = enc_e @ wab_r[...] + g_r[:, :H] + cg_r[...]
        if two_e:
            z = z + oute_r[...] @ wa_r[...]
        qb = jnp.broadcast_to(q_r[...][:, None, :], (NB_E, K, H))
        z = z + qb.reshape(NB_E * K, H)
        en = _sp(_sp(z) @ w2e_r[...] + b2e_r[...])
        enew_o[...] = en.reshape(NB_E * K // 2, 2, H)
        inc_o[...] = en.reshape(NB_E, K, H).sum(axis=1)
        s = en.sum(axis=0, keepdims=True)

        @pl.when(i == 0)
        def _():
            sume_o[...] = s

        @pl.when(i > 0)
        def _():
            sume_o[...] += s

    blk = NB_E * K
    in_specs = [
        pl.BlockSpec((blk, 4), lambda i: (i, 0)),
        pl.BlockSpec((blk, 2 * H), lambda i: (i, 0)),
        pl.BlockSpec((NB_E, H), lambda i: (i, 0)),
        _full((1, H)), _full((4, H)), _full((1, H)), _full((H, H)),
        _full((1, H)), _full((H, H)), _full((H, H)), _full((1, H)),
    ]
    args = [edges_flat, g_rows, q, cg, we1, be1, we2, be2, wab, w2e, b2e]
    if two_e:
        in_specs += [pl.BlockSpec((blk, H), lambda i: (i, 0)), _full((H, H))]
        args += [out_e, wa]
    out_shape = [
        jax.ShapeDtypeStruct((E // 2, 2, H), jnp.float32),
        jax.ShapeDtypeStruct((N, H), jnp.float32),
        jax.ShapeDtypeStruct((1, H), jnp.float32),
    ]
    out_specs = [
        pl.BlockSpec((blk // 2, 2, H), lambda i: (i, 0, 0)),
        pl.BlockSpec((NB_E, H), lambda i: (i, 0)),
        _full((1, H)),
    ]
    return pl.pallas_call(
        body, grid=(GRID_E,), in_specs=in_specs, out_specs=out_specs,
        out_shape=out_shape)(*args)


def _node_kernel(out_n, enc_n, inc, bnd, gnb, wna, wnb, wnc, wnd, w2n, b2n,
                 sum_e, ggb, wgn, wge, w2g, b2g, enc_g, nxt):
    final = len(nxt) == 6

    def body(*refs):
        (outn_r, encn_r, inc_r, ghi_r, glo_r, phi_r, plo_r, mhi_r, mlo_r,
         deg_r, gnb_r, wna_r, wnb_r, wnc_r,
         wnd_r, w2n_r, b2n_r, sume_r, ggb_r, wgn_r, wge_r, w2g_r, b2g_r,
         encg_r) = refs[:24]
        nxt_r = refs[24:24 + len(nxt)]
        outs = refs[24 + len(nxt):]
        i = pl.program_id(0)

        outg = (jnp.where(phi_r[...] > 0.5, ghi_r[:, H:], ghi_r[:, :H])
                * mhi_r[...]
                - jnp.where(plo_r[...] > 0.5, glo_r[:, H:], glo_r[:, :H])
                * mlo_r[...]
                + deg_r[...] * (sume_r[...] * (1.0 / E)))
        zn = (outn_r[...] @ wna_r[...] + encn_r[...] @ wnb_r[...]
              + inc_r[...] @ wnc_r[...]
              + outg @ wnd_r[...] + gnb_r[...])
        nn = _sp(_sp(zn) @ w2n_r[...] + b2n_r[...])
        if final:
            sumn_o, dec_o = outs
        else:
            (nnew_o, p2_o, q2_o, sumn_o, gnew_o, cg2_o, gnb2_o, ggb2_o) = outs
            nnew_o[...] = nn
            p2_o[:, :H] = nn @ nxt_r[0][...] + encn_r[...] @ nxt_r[1][...]
            p2_o[:, H:] = jnp.zeros((NB_N, H), jnp.float32)
            q2_o[...] = nn @ nxt_r[2][...] + encn_r[...] @ nxt_r[3][...]
        s = nn.sum(axis=0, keepdims=True)

        @pl.when(i == 0)
        def _():
            sumn_o[...] = s

        @pl.when(i > 0)
        def _():
            sumn_o[...] += s

        @pl.when(i == GRID_N - 1)
        def _():
            zg = (ggb_r[...] + sumn_o[...] @ wgn_r[...]
                  + sume_r[...] @ wge_r[...])
            gn = _sp(_sp(zg) @ w2g_r[...] + b2g_r[...])
            if final:
                d1, db1, d2, db2, d3, db3 = nxt_r
                d = _sp(gn @ d1[...] + db1[...])
                d = _sp(d @ d2[...] + db2[...])
                dec_o[...] = d @ d3[...] + db3[...]
            else:
                gnew_o[...] = gn
                cg2_o[...] = gn @ nxt_r[4][...] + encg_r[...] @ nxt_r[5][...] + nxt_r[6][...]
                gnb2_o[...] = gn @ nxt_r[7][...] + encg_r[...] @ nxt_r[8][...] + nxt_r[9][...]
                ggb2_o[...] = gn @ nxt_r[10][...] + encg_r[...] @ nxt_r[11][...] + nxt_r[12][...]

    nblk = pl.BlockSpec((NB_N, H), lambda i: (i, 0))
    gblk = pl.BlockSpec((NB_N, 2 * H), lambda i: (i, 0))
    pblk = pl.BlockSpec((NB_N, 1), lambda i: (i, 0))
    in_specs = [
        nblk, nblk, nblk,
        gblk, gblk, pblk, pblk, pblk, pblk, pblk,
        _full((1, H)), _full((H, H)), _full((H, H)), _full((H, H)),
        _full((H, H)), _full((H, H)), _full((1, H)),
        _full((1, H)), _full((1, H)), _full((H, H)), _full((H, H)),
        _full((H, H)), _full((1, H)), _full((1, H)),
    ]
    for w in nxt:
        in_specs.append(_full(tuple(w.shape)))
    if final:
        out_shape = [
            jax.ShapeDtypeStruct((1, H), jnp.float32),
            jax.ShapeDtypeStruct((1, DEC_PAD), jnp.float32),
        ]
        out_specs = [_full((1, H)), _full((1, DEC_PAD))]
    else:
        out_shape = [
            jax.ShapeDtypeStruct((N, H), jnp.float32),
            jax.ShapeDtypeStruct((N, 2 * H), jnp.float32),
            jax.ShapeDtypeStruct((N, H), jnp.float32),
            jax.ShapeDtypeStruct((1, H), jnp.float32),
            jax.ShapeDtypeStruct((1, H), jnp.float32),
            jax.ShapeDtypeStruct((1, H), jnp.float32),
            jax.ShapeDtypeStruct((1, H), jnp.float32),
            jax.ShapeDtypeStruct((1, H), jnp.float32),
        ]
        out_specs = ([nblk, pl.BlockSpec((NB_N, 2 * H), lambda i: (i, 0)),
                      nblk] + [_full((1, H))] * 5)
    ghi, glo, phi, plo, mhi, mlo, deg = bnd
    return pl.pallas_call(
        body, grid=(GRID_N,), in_specs=in_specs, out_specs=out_specs,
        out_shape=out_shape,
    )(out_n, enc_n, inc, ghi, glo, phi, plo, mhi, mlo, deg, gnb, wna, wnb,
      wnc, wnd, w2n, b2n, sum_e, ggb, wgn, wge, w2g, b2g, enc_g, *nxt)


def kernel(nodes, edges, globals_, edge_idx, params):
    f32 = jnp.float32
    i32 = jnp.int32
    idx3d = edge_idx.reshape(NCHUNK, SUB, 128)
    idx_flat = edge_idx.reshape(E)
    edges_flat = edges.reshape(E, 4)
    g2d = globals_.reshape(1, -1)

    perm = jnp.argsort(idx_flat)
    sorted_idx = idx_flat[perm]
    b = jnp.searchsorted(sorted_idx, jnp.arange(N + 1, dtype=i32)).astype(i32)
    gperm3d = (perm // 2).astype(i32).reshape(NCHUNK, SUB, 128)
    spar = (perm % 2).astype(f32).reshape(E, 1)
    lo = jnp.maximum(b[:-1] - 1, 0)
    hi = jnp.maximum(b[1:] - 1, 0)
    PADN = 16384
    lo3d = jnp.zeros((PADN,), i32).at[:N].set(lo // 2).reshape(
        PADN // CH, SUB, 128)
    hi3d = jnp.zeros((PADN,), i32).at[:N].set(hi // 2).reshape(
        PADN // CH, SUB, 128)
    plo = (lo % 2).astype(f32).reshape(N, 1)
    phi = (hi % 2).astype(f32).reshape(N, 1)
    mlo = (b[:-1] > 0).astype(f32).reshape(N, 1)
    mhi = (b[1:] > 0).astype(f32).reshape(N, 1)
    deg = (b[1:] - b[:-1]).astype(f32).reshape(N, 1)
    tri = (jnp.arange(NB_E)[:, None] > jnp.arange(NB_E)[None, :]).astype(f32)

    def outgoing_bnd(e3, sume):
        srt = _sc_gather(e3.reshape(E // 2, 2 * H), gperm3d)
        s = _cumsum_kernel(srt, spar, sume, tri)
        stab = s.reshape(E // 2, 2 * H)
        return (_sc_gather(stab, hi3d), _sc_gather(stab, lo3d),
                phi, plo, mhi, mlo, deg)

    def row2d(b):
        return b.reshape(1, -1).astype(f32)

    (wn1, bn1), (wn2, bn2) = params['node_enc']
    (we1, be1), (we2, be2) = params['edge_enc']
    (wg1, bg1), (wg2, bg2) = params['glob_enc']
    bn1, bn2, bg1, bg2, be1, be2 = map(row2d, (bn1, bn2, bg1, bg2, be1, be2))

    rblk = []
    for rp in params['rec']:
        (w1e, b1e), (w2e, b2e) = rp['edge_fn']
        (w1n, b1n), (w2n, b2n) = rp['node_fn']
        (w1g, b1g), (w2g, b2g) = rp['glob_fn']
        rblk.append(dict(
            wa=w1e[0:64], wb=w1e[64:128],
            wp_a=w1e[128:192], wp_b=w1e[192:256],
            wq_a=w1e[256:320], wq_b=w1e[320:384],
            wcg_a=w1e[384:448], wcg_b=w1e[448:512],
            b1e=row2d(b1e), w2e=w2e, b2e=row2d(b2e),
            wna=w1n[0:64], wnb=w1n[64:128], wnc=w1n[128:192],
            wnd=w1n[192:256], wgnb_a=w1n[256:320], wgnb_b=w1n[320:384],
            b1n=row2d(b1n), w2n=w2n, b2n=row2d(b2n),
            wgg_a=w1g[0:64], wgg_b=w1g[64:128], wgn=w1g[128:192],
            wge=w1g[192:256], b1g=row2d(b1g), w2g=w2g, b2g=row2d(b2g)))
    r1, r2 = rblk

    (d1, db1), (d2, db2), (d3, db3) = params['decoder']
    d3p = jnp.zeros((H, DEC_PAD), f32).at[:, :d3.shape[1]].set(d3)
    db3p = jnp.zeros((1, DEC_PAD), f32).at[0, :d3.shape[1]].set(db3)

    enc_n, p1, q1, enc_g, cg1, gnb1, ggb1 = _prep_kernel(
        nodes, g2d, wn1, bn1, wn2, bn2, wg1, bg1, wg2, bg2,
        r1['wp_a'] + r1['wp_b'], r1['wq_a'] + r1['wq_b'],
        r1['wcg_a'] + r1['wcg_b'], r1['b1e'],
        r1['wgnb_a'] + r1['wgnb_b'], r1['b1n'],
        r1['wgg_a'] + r1['wgg_b'], r1['b1g'])

    g1_rows = _sc_gather(p1, idx3d)
    e_new1, inc1, sum_e1 = _edge_kernel(
        edges_flat, g1_rows, q1, cg1, we1, be1, we2, be2,
        r1['wa'] + r1['wb'], r1['w2e'], r1['b2e'])
    bnd1 = outgoing_bnd(e_new1, sum_e1)
    nxt = (r2['wp_a'], r2['wp_b'], r2['wq_a'], r2['wq_b'],
           r2['wcg_a'], r2['wcg_b'], r2['b1e'],
           r2['wgnb_a'], r2['wgnb_b'], r2['b1n'],
           r2['wgg_a'], r2['wgg_b'], r2['b1g'])
    (n_new1, p2, q2, _sum_n1, g_new1, cg2, gnb2, ggb2) = _node_kernel(
        enc_n, enc_n, inc1, bnd1, gnb1,
        r1['wna'] + r1['wnb'], jnp.zeros((H, H), f32), r1['wnc'], r1['wnd'],
        r1['w2n'], r1['b2n'], sum_e1, ggb1, r1['wgn'], r1['wge'],
        r1['w2g'], r1['b2g'], enc_g, nxt)

    g2_rows = _sc_gather(p2, idx3d)
    e_new2, inc2, sum_e2 = _edge_kernel(
        edges_flat, g2_rows, q2, cg2, we1, be1, we2, be2,
        r2['wb'], r2['w2e'], r2['b2e'],
        out_e=e_new1.reshape(E, H), wa=r2['wa'])
    bnd2 = outgoing_bnd(e_new2, sum_e2)
    dec = (d1, row2d(db1), d2, row2d(db2), d3p, db3p)
    _sum_n2, out = _node_kernel(
        n_new1, enc_n, inc2, bnd2, gnb2,
        r2['wna'], r2['wnb'], r2['wnc'], r2['wnd'],
        r2['w2n'], r2['b2n'], sum_e2, ggb2, r2['wgn'], r2['wge'],
        r2['w2g'], r2['b2g'], enc_g, dec)

    return out[0, :d3.shape[1]]


# speedup vs baseline: 1.9682x; 1.7593x over previous
import functools

import jax
import jax.numpy as jnp
from jax import lax
from jax.experimental import pallas as pl
from jax.experimental.pallas import tpu as pltpu
from jax.experimental.pallas import tpu_sc as plsc

N = 10000
K = 32
E = N * K
H = 64
NB_E = 200
GRID_E = N // NB_E
NB_N = 2000
GRID_N = N // NB_N
CH = 512
NCHUNK = E // CH
SUB = CH // 128
NW = 32
PER_W = -(-NCHUNK // NW)
DEC_PAD = 512


def _sp(x):
    return jnp.maximum(x, 0.0) + jnp.log1p(jnp.exp(-jnp.abs(x)))


def _sc_gather(table, idx3d):
    mesh = plsc.VectorSubcoreMesh(core_axis_name="c", subcore_axis_name="s")
    nch = idx3d.shape[0]
    m = nch * CH
    trips = -(-nch // NW)

    @functools.partial(
        pl.kernel, mesh=mesh,
        out_type=jax.ShapeDtypeStruct((m, 2 * H), jnp.float32),
        scratch_types=[
            pltpu.VMEM((SUB, 128), jnp.int32),
            pltpu.VMEM((CH, 2 * H), jnp.float32),
            pltpu.SemaphoreType.DMA,
        ],
    )
    def k(table_hbm, idx_hbm, out_hbm, idx_v, rows_v, sem):
        wid = lax.axis_index("s") * 2 + lax.axis_index("c")

        def body(t, carry):
            cid = wid + NW * t

            @pl.when(cid < nch)
            def _():
                pltpu.sync_copy(idx_hbm.at[cid], idx_v)
                cps = [
                    pltpu.async_copy(table_hbm.at[idx_v.at[j]],
                                     rows_v.at[pl.ds(j * 128, 128)], sem)
                    for j in range(SUB)
                ]
                for cp in cps:
                    cp.wait()
                pltpu.sync_copy(rows_v, out_hbm.at[pl.ds(cid * CH, CH)])

            return carry

        lax.fori_loop(0, trips, body, 0)

    return k(table, idx3d)


def _cumsum_kernel(srt, par, sume, tri):
    blk = NB_E * K

    def body(srt_r, par_r, sume_r, tri_r, s_o, carry):
        i = pl.program_id(0)

        @pl.when(i == 0)
        def _():
            carry[...] = jnp.zeros((1, H), jnp.float32)

        sel = (jnp.where(par_r[...] > 0.5, srt_r[:, H:], srt_r[:, :H])
               - sume_r[...] * (1.0 / E))
        x3 = sel.reshape(blk // 32, 32, H)
        f = x3
        sh = 1
        while sh < 32:
            f = f + jnp.concatenate(
                [jnp.zeros((blk // 32, sh, H), jnp.float32), f[:, :-sh]],
                axis=1)
            sh *= 2
        coarse = tri_r[...] @ x3.sum(axis=1)
        s = (f + coarse[:, None, :]).reshape(blk, H) + carry[...]
        s_o[...] = s
        carry[...] = s[blk - 1:blk, :]

    return pl.pallas_call(
        body, grid=(GRID_E,),
        in_specs=[pl.BlockSpec((blk, 2 * H), lambda i: (i, 0)),
                  pl.BlockSpec((blk, 1), lambda i: (i, 0)),
                  _full((1, H)), _full((NB_E, NB_E))],
        out_specs=pl.BlockSpec((blk, H), lambda i: (i, 0)),
        out_shape=jax.ShapeDtypeStruct((E, H), jnp.float32),
        scratch_shapes=[pltpu.VMEM((1, H), jnp.float32)],
    )(srt, par, sume, tri)


def _full(shape):
    return pl.BlockSpec(shape, lambda i: tuple(0 for _ in shape))


def _prep_kernel(nodes, g2d, wn1, bn1, wn2, bn2, wg1, bg1, wg2, bg2,
                 wp1, wq1, wcg1, b1e1, wgnb1, b1n1, wggb1, b1g1):
    def body(nodes_r, g_r, wn1_r, bn1_r, wn2_r, bn2_r, wg1_r, bg1_r, wg2_r,
             bg2_r, wp1_r, wq1_r, wcg1_r, b1e1_r, wgnb1_r, b1n1_r, wggb1_r,
             b1g1_r, enc_n_o, p1_o, q1_o, enc_g_o, cg1_o, gnb1_o, ggb1_o):
        enc_n = _sp(_sp(nodes_r[...] @ wn1_r[...] + bn1_r[...])
                    @ wn2_r[...] + bn2_r[...])
        enc_g = _sp(_sp(g_r[...] @ wg1_r[...] + bg1_r[...])
                    @ wg2_r[...] + bg2_r[...])
        enc_n_o[...] = enc_n
        p1_o[:, :H] = enc_n @ wp1_r[...]
        p1_o[:, H:] = jnp.zeros((N, H), jnp.float32)
        q1_o[...] = enc_n @ wq1_r[...]
        enc_g_o[...] = enc_g
        cg1_o[...] = enc_g @ wcg1_r[...] + b1e1_r[...]
        gnb1_o[...] = enc_g @ wgnb1_r[...] + b1n1_r[...]
        ggb1_o[...] = enc_g @ wggb1_r[...] + b1g1_r[...]

    out_shape = [
        jax.ShapeDtypeStruct((N, H), jnp.float32),
        jax.ShapeDtypeStruct((N, 2 * H), jnp.float32),
        jax.ShapeDtypeStruct((N, H), jnp.float32),
        jax.ShapeDtypeStruct((1, H), jnp.float32),
        jax.ShapeDtypeStruct((1, H), jnp.float32),
        jax.ShapeDtypeStruct((1, H), jnp.float32),
        jax.ShapeDtypeStruct((1, H), jnp.float32),
    ]
    return pl.pallas_call(body, out_shape=out_shape)(
        nodes, g2d, wn1, bn1, wn2, bn2, wg1, bg1, wg2, bg2,
        wp1, wq1, wcg1, b1e1, wgnb1, b1n1, wggb1, b1g1)


def _edge_kernel(edges_flat, g_rows, q, cg, we1, be1, we2, be2,
                 wab, w2e, b2e, out_e=None, wa=None):
    two_e = out_e is not None

    def body(*refs):
        if two_e:
            (edges_r, g_r, q_r, cg_r, we1_r, be1_r, we2_r, be2_r, wab_r,
             w2e_r, b2e_r, oute_r, wa_r, enew_o, inc_o, sume_o) = refs
        else:
            (edges_r, g_r, q_r, cg_r, we1_r, be1_r, we2_r, be2_r, wab_r,
             w2e_r, b2e_r, enew_o, inc_o, sume_o) = refs
        i = pl.program_id(0)
        enc_e = _sp(_sp(edges_r[...] @ we1_r[...] + be1_r[...])
                    @ we2_r[...] + be2_r[...])
        z = enc_e @ wab_r[...] + g_r[:, :H] + cg_r[...]
        if two_e:
            z = z + oute_r[...] @ wa_r[...]
        qb = jnp.broadcast_to(q_r[...][:, None, :], (NB_E, K, H))
        z = z + qb.reshape(NB_E * K, H)
        en = _sp(_sp(z) @ w2e_r[...] + b2e_r[...])
        enew_o[...] = en.reshape(NB_E * K // 2, 2, H)
        inc_o[...] = en.reshape(NB_E, K, H).sum(axis=1)
        s = en.sum(axis=0, keepdims=True)

        @pl.when(i == 0)
        def _():
            sume_o[...] = s

        @pl.when(i > 0)
        def _():
            sume_o[...] += s

    blk = NB_E * K
    in_specs = [
        pl.BlockSpec((blk, 4), lambda i: (i, 0)),
        pl.BlockSpec((blk, 2 * H), lambda i: (i, 0)),
        pl.BlockSpec((NB_E, H), lambda i: (i, 0)),
        _full((1, H)), _full((4, H)), _full((1, H)), _full((H, H)),
        _full((1, H)), _full((H, H)), _full((H, H)), _full((1, H)),
    ]
    args = [edges_flat, g_rows, q, cg, we1, be1, we2, be2, wab, w2e, b2e]
    if two_e:
        in_specs += [pl.BlockSpec((blk, H), lambda i: (i, 0)), _full((H, H))]
        args += [out_e, wa]
    out_shape = [
        jax.ShapeDtypeStruct((E // 2, 2, H), jnp.float32),
        jax.ShapeDtypeStruct((N, H), jnp.float32),
        jax.ShapeDtypeStruct((1, H), jnp.float32),
    ]
    out_specs = [
        pl.BlockSpec((blk // 2, 2, H), lambda i: (i, 0, 0)),
        pl.BlockSpec((NB_E, H), lambda i: (i, 0)),
        _full((1, H)),
    ]
    return pl.pallas_call(
        body, grid=(GRID_E,), in_specs=in_specs, out_specs=out_specs,
        out_shape=out_shape)(*args)


def _node_kernel(out_n, enc_n, inc, bnd, gnb, wna, wnb, wnc, wnd, w2n, b2n,
                 sum_e, ggb, wgn, wge, w2g, b2g, enc_g, nxt):
    final = len(nxt) == 6

    def body(*refs):
        (outn_r, encn_r, inc_r, ghi_r, glo_r, phi_r, plo_r, mhi_r, mlo_r,
         deg_r, gnb_r, wna_r, wnb_r, wnc_r,
         wnd_r, w2n_r, b2n_r, sume_r, ggb_r, wgn_r, wge_r, w2g_r, b2g_r,
         encg_r) = refs[:24]
        nxt_r = refs[24:24 + len(nxt)]
        outs = refs[24 + len(nxt):]
        i = pl.program_id(0)

        outg = (jnp.where(phi_r[...] > 0.5, ghi_r[:, H:], ghi_r[:, :H])
                * mhi_r[...]
                - jnp.where(plo_r[...] > 0.5, glo_r[:, H:], glo_r[:, :H])
                * mlo_r[...]
                + deg_r[...] * (sume_r[...] * (1.0 / E)))
        zn = (outn_r[...] @ wna_r[...] + encn_r[...] @ wnb_r[...]
              + inc_r[...] @ wnc_r[...]
              + outg @ wnd_r[...] + gnb_r[...])
        nn = _sp(_sp(zn) @ w2n_r[...] + b2n_r[...])
        if final:
            sumn_o, dec_o = outs
        else:
            (nnew_o, p2_o, q2_o, sumn_o, gnew_o, cg2_o, gnb2_o, ggb2_o) = outs
            nnew_o[...] = nn
            p2_o[:, :H] = nn @ nxt_r[0][...] + encn_r[...] @ nxt_r[1][...]
            p2_o[:, H:] = jnp.zeros((NB_N, H), jnp.float32)
            q2_o[...] = nn @ nxt_r[2][...] + encn_r[...] @ nxt_r[3][...]
        s = nn.sum(axis=0, keepdims=True)

        @pl.when(i == 0)
        def _():
            sumn_o[...] = s

        @pl.when(i > 0)
        def _():
            sumn_o[...] += s

        @pl.when(i == GRID_N - 1)
        def _():
            zg = (ggb_r[...] + sumn_o[...] @ wgn_r[...]
                  + sume_r[...] @ wge_r[...])
            gn = _sp(_sp(zg) @ w2g_r[...] + b2g_r[...])
            if final:
                d1, db1, d2, db2, d3, db3 = nxt_r
                d = _sp(gn @ d1[...] + db1[...])
                d = _sp(d @ d2[...] + db2[...])
                dec_o[...] = d @ d3[...] + db3[...]
            else:
                gnew_o[...] = gn
                cg2_o[...] = gn @ nxt_r[4][...] + encg_r[...] @ nxt_r[5][...] + nxt_r[6][...]
                gnb2_o[...] = gn @ nxt_r[7][...] + encg_r[...] @ nxt_r[8][...] + nxt_r[9][...]
                ggb2_o[...] = gn @ nxt_r[10][...] + encg_r[...] @ nxt_r[11][...] + nxt_r[12][...]

    nblk = pl.BlockSpec((NB_N, H), lambda i: (i, 0))
    gblk = pl.BlockSpec((NB_N, 2 * H), lambda i: (i, 0))
    pblk = pl.BlockSpec((NB_N, 1), lambda i: (i, 0))
    in_specs = [
        nblk, nblk, nblk,
        gblk, gblk, pblk, pblk, pblk, pblk, pblk,
        _full((1, H)), _full((H, H)), _full((H, H)), _full((H, H)),
        _full((H, H)), _full((H, H)), _full((1, H)),
        _full((1, H)), _full((1, H)), _full((H, H)), _full((H, H)),
        _full((H, H)), _full((1, H)), _full((1, H)),
    ]
    for w in nxt:
        in_specs.append(_full(tuple(w.shape)))
    if final:
        out_shape = [
            jax.ShapeDtypeStruct((1, H), jnp.float32),
            jax.ShapeDtypeStruct((1, DEC_PAD), jnp.float32),
        ]
        out_specs = [_full((1, H)), _full((1, DEC_PAD))]
    else:
        out_shape = [
            jax.ShapeDtypeStruct((N, H), jnp.float32),
            jax.ShapeDtypeStruct((N, 2 * H), jnp.float32),
            jax.ShapeDtypeStruct((N, H), jnp.float32),
            jax.ShapeDtypeStruct((1, H), jnp.float32),
            jax.ShapeDtypeStruct((1, H), jnp.float32),
            jax.ShapeDtypeStruct((1, H), jnp.float32),
            jax.ShapeDtypeStruct((1, H), jnp.float32),
            jax.ShapeDtypeStruct((1, H), jnp.float32),
        ]
        out_specs = ([nblk, pl.BlockSpec((NB_N, 2 * H), lambda i: (i, 0)),
                      nblk] + [_full((1, H))] * 5)
    ghi, glo, phi, plo, mhi, mlo, deg = bnd
    return pl.pallas_call(
        body, grid=(GRID_N,), in_specs=in_specs, out_specs=out_specs,
        out_shape=out_shape,
    )(out_n, enc_n, inc, ghi, glo, phi, plo, mhi, mlo, deg, gnb, wna, wnb,
      wnc, wnd, w2n, b2n, sum_e, ggb, wgn, wge, w2g, b2g, enc_g, *nxt)


def kernel(nodes, edges, globals_, edge_idx, params):
    f32 = jnp.float32
    i32 = jnp.int32
    idx3d = edge_idx.reshape(NCHUNK, SUB, 128)
    idx_flat = edge_idx.reshape(E)
    edges_flat = edges.reshape(E, 4)
    g2d = globals_.reshape(1, -1)

    sorted_idx, perm = lax.sort(
        (idx_flat, jnp.arange(E, dtype=jnp.int32)), num_keys=1)
    b = jnp.searchsorted(sorted_idx, jnp.arange(N + 1, dtype=i32)).astype(i32)
    gperm3d = (perm // 2).astype(i32).reshape(NCHUNK, SUB, 128)
    spar = (perm % 2).astype(f32).reshape(E, 1)
    lo = jnp.maximum(b[:-1] - 1, 0)
    hi = jnp.maximum(b[1:] - 1, 0)
    PADN = 16384
    lo3d = jnp.zeros((PADN,), i32).at[:N].set(lo // 2).reshape(
        PADN // CH, SUB, 128)
    hi3d = jnp.zeros((PADN,), i32).at[:N].set(hi // 2).reshape(
        PADN // CH, SUB, 128)
    plo = (lo % 2).astype(f32).reshape(N, 1)
    phi = (hi % 2).astype(f32).reshape(N, 1)
    mlo = (b[:-1] > 0).astype(f32).reshape(N, 1)
    mhi = (b[1:] > 0).astype(f32).reshape(N, 1)
    deg = (b[1:] - b[:-1]).astype(f32).reshape(N, 1)
    tri = (jnp.arange(NB_E)[:, None] > jnp.arange(NB_E)[None, :]).astype(f32)

    def outgoing_bnd(e3, sume):
        srt = _sc_gather(e3.reshape(E // 2, 2 * H), gperm3d)
        s = _cumsum_kernel(srt, spar, sume, tri)
        stab = s.reshape(E // 2, 2 * H)
        return (_sc_gather(stab, hi3d), _sc_gather(stab, lo3d),
                phi, plo, mhi, mlo, deg)

    def row2d(b):
        return b.reshape(1, -1).astype(f32)

    (wn1, bn1), (wn2, bn2) = params['node_enc']
    (we1, be1), (we2, be2) = params['edge_enc']
    (wg1, bg1), (wg2, bg2) = params['glob_enc']
    bn1, bn2, bg1, bg2, be1, be2 = map(row2d, (bn1, bn2, bg1, bg2, be1, be2))

    rblk = []
    for rp in params['rec']:
        (w1e, b1e), (w2e, b2e) = rp['edge_fn']
        (w1n, b1n), (w2n, b2n) = rp['node_fn']
        (w1g, b1g), (w2g, b2g) = rp['glob_fn']
        rblk.append(dict(
            wa=w1e[0:64], wb=w1e[64:128],
            wp_a=w1e[128:192], wp_b=w1e[192:256],
            wq_a=w1e[256:320], wq_b=w1e[320:384],
            wcg_a=w1e[384:448], wcg_b=w1e[448:512],
            b1e=row2d(b1e), w2e=w2e, b2e=row2d(b2e),
            wna=w1n[0:64], wnb=w1n[64:128], wnc=w1n[128:192],
            wnd=w1n[192:256], wgnb_a=w1n[256:320], wgnb_b=w1n[320:384],
            b1n=row2d(b1n), w2n=w2n, b2n=row2d(b2n),
            wgg_a=w1g[0:64], wgg_b=w1g[64:128], wgn=w1g[128:192],
            wge=w1g[192:256], b1g=row2d(b1g), w2g=w2g, b2g=row2d(b2g)))
    r1, r2 = rblk

    (d1, db1), (d2, db2), (d3, db3) = params['decoder']
    d3p = jnp.zeros((H, DEC_PAD), f32).at[:, :d3.shape[1]].set(d3)
    db3p = jnp.zeros((1, DEC_PAD), f32).at[0, :d3.shape[1]].set(db3)

    enc_n, p1, q1, enc_g, cg1, gnb1, ggb1 = _prep_kernel(
        nodes, g2d, wn1, bn1, wn2, bn2, wg1, bg1, wg2, bg2,
        r1['wp_a'] + r1['wp_b'], r1['wq_a'] + r1['wq_b'],
        r1['wcg_a'] + r1['wcg_b'], r1['b1e'],
        r1['wgnb_a'] + r1['wgnb_b'], r1['b1n'],
        r1['wgg_a'] + r1['wgg_b'], r1['b1g'])

    g1_rows = _sc_gather(p1, idx3d)
    e_new1, inc1, sum_e1 = _edge_kernel(
        edges_flat, g1_rows, q1, cg1, we1, be1, we2, be2,
        r1['wa'] + r1['wb'], r1['w2e'], r1['b2e'])
    bnd1 = outgoing_bnd(e_new1, sum_e1)
    nxt = (r2['wp_a'], r2['wp_b'], r2['wq_a'], r2['wq_b'],
           r2['wcg_a'], r2['wcg_b'], r2['b1e'],
           r2['wgnb_a'], r2['wgnb_b'], r2['b1n'],
           r2['wgg_a'], r2['wgg_b'], r2['b1g'])
    (n_new1, p2, q2, _sum_n1, g_new1, cg2, gnb2, ggb2) = _node_kernel(
        enc_n, enc_n, inc1, bnd1, gnb1,
        r1['wna'] + r1['wnb'], jnp.zeros((H, H), f32), r1['wnc'], r1['wnd'],
        r1['w2n'], r1['b2n'], sum_e1, ggb1, r1['wgn'], r1['wge'],
        r1['w2g'], r1['b2g'], enc_g, nxt)

    g2_rows = _sc_gather(p2, idx3d)
    e_new2, inc2, sum_e2 = _edge_kernel(
        edges_flat, g2_rows, q2, cg2, we1, be1, we2, be2,
        r2['wb'], r2['w2e'], r2['b2e'],
        out_e=e_new1.reshape(E, H), wa=r2['wa'])
    bnd2 = outgoing_bnd(e_new2, sum_e2)
    dec = (d1, row2d(db1), d2, row2d(db2), d3p, db3p)
    _sum_n2, out = _node_kernel(
        n_new1, enc_n, inc2, bnd2, gnb2,
        r2['wna'], r2['wnb'], r2['wnc'], r2['wnd'],
        r2['w2n'], r2['b2n'], sum_e2, ggb2, r2['wgn'], r2['wge'],
        r2['w2g'], r2['b2g'], enc_g, dec)

    return out[0, :d3.shape[1]]
